# baseline (speedup 1.0000x reference)
import jax
import jax.numpy as jnp
from jax import lax
from jax.experimental import pallas as pl
from jax.experimental.pallas import tpu as pltpu
from jax.experimental.pallas import tpu_sc as plsc

N = 20000
G = 128
C = 80
NP = 20480
TB = 2048
NW = 32
RPW = 640
CHUNKS = RPW // 16


def _tc1_body(rx0_ref, ry0_ref, rx1_ref, ry1_ref, tgt_ref,
              cx_ref, cy_ref, gcx_ref, gcy_ref, glb_ref):
    tgtt = jnp.transpose(tgt_ref[...])
    cx_ref[pl.ds(0, N)] = (rx0_ref[...] + rx1_ref[...]) / 2.0
    cy_ref[pl.ds(0, N)] = (ry0_ref[...] + ry1_ref[...]) / 2.0
    gcx_ref[...] = (tgtt[0] + tgtt[2]) / 2.0
    gcy_ref[...] = (tgtt[1] + tgtt[3]) / 2.0
    glb_ref[...] = tgtt[4]


def _tc2_body(cls_ref, sidx_ref):
    clst = jnp.transpose(cls_ref[...])
    maxv = jnp.max(clst, axis=0)
    ciota = lax.broadcasted_iota(jnp.int32, clst.shape, 0)
    cidx = jnp.min(jnp.where(clst == maxv[None, :], ciota, C), axis=0)
    sidx_ref[...] = jnp.where(maxv > 0.0, cidx, -1)


def _tc3_body(bidx_ref, d2_ref, glb_ref, sidx_ref, asg_ref, dis_ref, lbl_ref):
    bi = bidx_ref[...]
    glab_i = glb_ref[...]
    pos = sidx_ref[...] == glab_i
    asg_ref[...] = jnp.where(pos, bi + 1, 0)
    dis_ref[...] = jnp.sqrt(d2_ref[...])
    lbl_ref[...] = jnp.where(pos, glab_i, -1)


def _sc_body(cx_hbm, cy_hbm, gcx_hbm, gcy_hbm, glb_hbm,
             bidx_hbm, d2_hbm, glbo_hbm,
             cx_v, cy_v, gcx_v, gcy_v, glb_v,
             bidx_v, d2_v, glbo_v, sem):
    wid = lax.axis_index("s") * 2 + lax.axis_index("c")
    base = jnp.minimum(wid * RPW, N - RPW)

    iota = jnp.arange(16, dtype=jnp.int32)

    cps = [
        pltpu.async_copy(cx_hbm.at[pl.ds(base, RPW)], cx_v, sem),
        pltpu.async_copy(cy_hbm.at[pl.ds(base, RPW)], cy_v, sem),
        pltpu.async_copy(gcx_hbm, gcx_v, sem),
        pltpu.async_copy(gcy_hbm, gcy_v, sem),
        pltpu.async_copy(glb_hbm, glb_v, sem),
    ]
    for cp in cps:
        cp.wait()

    NH = 2

    def chunk(j, carry):
        rows_h, cx_h, cy_h = [], [], []
        for h in range(NH):
            rows = iota + (j * (16 * NH) + 16 * h)
            rows_h.append(rows)
            cx_h.append(plsc.load_gather(cx_v, [rows]))
            cy_h.append(plsc.load_gather(cy_v, [rows]))

        inf16 = jnp.full((16,), jnp.inf, jnp.float32)
        zero16 = jnp.zeros((16,), jnp.int32)

        def kbody(k, c):
            best, bidx, gv = c
            gx16 = gcx_v[pl.ds(k * 16, 16)]
            gy16 = gcy_v[pl.ds(k * 16, 16)]
            for jj in range(16):
                gx = gx16[jj]
                gy = gy16[jj]
                for h in range(NH):
                    dx = cx_h[h] - gx
                    dy = cy_h[h] - gy
                    d2 = dx * dx + dy * dy
                    m = d2 < best[h]
                    best[h] = jnp.where(m, d2, best[h])
                    bidx[h] = jnp.where(m, gv, bidx[h])
                gv = gv + 1
            return (best, bidx, gv)

        best, bidx, _ = lax.fori_loop(
            0, G // 16, kbody,
            ([inf16] * NH, [zero16] * NH, zero16))

        for h in range(NH):
            b, bi = best[h], bidx[h]
            glab = plsc.load_gather(glb_v, [bi])

            plsc.store_scatter(bidx_v, [rows_h[h]], bi)
            plsc.store_scatter(d2_v, [rows_h[h]], b)
            plsc.store_scatter(glbo_v, [rows_h[h]], glab.astype(jnp.int32))
        return carry

    lax.fori_loop(0, CHUNKS // NH, chunk, 0)

    pltpu.sync_copy(bidx_v, bidx_hbm.at[pl.ds(base, RPW)])
    pltpu.sync_copy(d2_v, d2_hbm.at[pl.ds(base, RPW)])
    pltpu.sync_copy(glbo_v, glbo_hbm.at[pl.ds(base, RPW)])


@jax.jit
def _run(reg_pred, targets, cls_pred):
    nb = NP // TB
    rx0 = reg_pred[:, 0]
    ry0 = reg_pred[:, 1]
    rx1 = reg_pred[:, 2]
    ry1 = reg_pred[:, 3]
    cx, cy, gcx, gcy, glb = pl.pallas_call(
        _tc1_body,
        out_shape=(
            jax.ShapeDtypeStruct((NP,), jnp.float32),
            jax.ShapeDtypeStruct((NP,), jnp.float32),
            jax.ShapeDtypeStruct((G,), jnp.float32),
            jax.ShapeDtypeStruct((G,), jnp.float32),
            jax.ShapeDtypeStruct((G,), jnp.float32),
        ),
    )(rx0, ry0, rx1, ry1, targets)

    mesh = plsc.VectorSubcoreMesh(core_axis_name="c", subcore_axis_name="s")
    sc = pl.kernel(
        _sc_body,
        mesh=mesh,
        compiler_params=pltpu.CompilerParams(needs_layout_passes=False),
        out_type=(
            jax.ShapeDtypeStruct((N,), jnp.int32),
            jax.ShapeDtypeStruct((N,), jnp.float32),
            jax.ShapeDtypeStruct((N,), jnp.int32),
        ),
        scratch_types=[
            pltpu.VMEM((RPW,), jnp.float32),
            pltpu.VMEM((RPW,), jnp.float32),
            pltpu.VMEM((G,), jnp.float32),
            pltpu.VMEM((G,), jnp.float32),
            pltpu.VMEM((G,), jnp.float32),
            pltpu.VMEM((RPW,), jnp.int32),
            pltpu.VMEM((RPW,), jnp.float32),
            pltpu.VMEM((RPW,), jnp.int32),
            pltpu.SemaphoreType.DMA,
        ],
    )
    bidx, d2m, glbi = sc(cx, cy, gcx, gcy, glb)

    sidx = pl.pallas_call(
        _tc2_body,
        grid=(nb,),
        in_specs=[pl.BlockSpec((TB, C), lambda i: (i, 0))],
        out_specs=pl.BlockSpec((TB,), lambda i: (i,)),
        out_shape=jax.ShapeDtypeStruct((NP,), jnp.int32),
        compiler_params=pltpu.CompilerParams(vmem_limit_bytes=6 * 1024 * 1024),
    )(cls_pred)

    asg, dis, lbl = pl.pallas_call(
        _tc3_body,
        out_shape=(
            jax.ShapeDtypeStruct((N,), jnp.int32),
            jax.ShapeDtypeStruct((N,), jnp.float32),
            jax.ShapeDtypeStruct((N,), jnp.int32),
        ),
    )(bidx, d2m, glbi, sidx[:N])
    return asg, dis, lbl


def kernel(reg_pred, targets, num_level_bboxes, cls_pred):
    asg, dis, lbl = _run(reg_pred, targets, cls_pred)
    return (asg, dis, lbl, reg_pred, targets)
